# baseline (speedup 1.0000x reference)
import numpy as np
import jax
import jax.numpy as jnp
from jax.experimental import pallas as pl
from jax.experimental.pallas import tpu as pltpu

_PATCH = 8
_IMGS_PER_STEP = 4


def _extract_body(x_ref, x2_ref, m_ref, wt_ref, s4_ref, rp4_ref, rp2_ref,
                  s2_ref, oi_ref, om_ref):
    n_lat, n_ch = wt_ref.shape[0], wt_ref.shape[1]
    H, W = x_ref.shape[2], x_ref.shape[3]
    Wm = s4_ref.shape[1]
    Hm = rp4_ref.shape[0]
    Hl = rp2_ref.shape[0]
    hs = H // _PATCH

    half = x_ref.shape[0]
    for g in range(2 * half):
        xr = x_ref if g < half else x2_ref
        xb = [xr[g % half, c].astype(jnp.bfloat16).reshape(hs, _PATCH, W)
              for c in range(n_ch)]
        mb = m_ref[g].astype(jnp.bfloat16)

        ts = []
        for n in range(n_lat):
            y = xb[0] * wt_ref[n, 0][None, :, :]
            for c in range(1, n_ch):
                y = y + xb[c] * wt_ref[n, c][None, :, :]
            ts.append(jnp.dot(rp4_ref[...], y.reshape(H, W),
                              preferred_element_type=jnp.float32))
        t4 = jnp.concatenate(ts, axis=0).astype(jnp.bfloat16)

        z = jnp.dot(t4, s4_ref[...],
                    preferred_element_type=jnp.float32)
        zb = z.astype(jnp.bfloat16)

        pieces = []
        for n in range(n_lat):
            zn = zb[n * Hm:(n + 1) * Hm]
            pieces.append(zn)
            pieces.append(zn * mb)
        tall = jnp.concatenate(pieces, axis=1)
        t2 = jnp.dot(rp2_ref[...], tall,
                     preferred_element_type=jnp.float32)

        s = jnp.concatenate(
            [t2[:, k * Wm:(k + 1) * Wm] for k in range(2 * n_lat)], axis=0)
        o = jnp.dot(s, s2_ref[...],
                    preferred_element_type=jnp.float32)

        for n in range(n_lat):
            oi_ref[g, n] = o[2 * n * Hl:(2 * n + 1) * Hl].astype(oi_ref.dtype)
            om_ref[g, n] = o[(2 * n + 1) * Hl:
                             (2 * n + 2) * Hl].astype(om_ref.dtype)


def kernel(images, ref_masks, w_kernel):
    B, C, H, W = images.shape
    _, Hm, Wm = ref_masks.shape
    N = w_kernel.shape[0]
    Hl, Wl = H // _PATCH, W // _PATCH
    qh, qw = H // Hm, W // Wm
    cph = _PATCH // qh
    cpw = _PATCH // qw
    dt = images.dtype
    bf = jnp.bfloat16

    t8 = np.equal(np.arange(W)[None, :] % _PATCH,
                  np.arange(_PATCH)[:, None]).astype(np.float32)
    wt = jnp.einsum('ncjd,dw->ncjw',
                    w_kernel.reshape(N, C, _PATCH, _PATCH), t8,
                    precision=jax.lax.Precision.HIGHEST).astype(bf)

    s4 = jnp.asarray(np.equal(np.arange(W)[:, None] // qw,
                              np.arange(Wm)[None, :]), dtype=bf)
    rp4 = jnp.asarray(np.equal(np.arange(H)[None, :] // qh,
                               np.arange(Hm)[:, None]), dtype=bf)
    rp2 = jnp.asarray(np.equal(np.arange(Hm)[None, :] // cph,
                               np.arange(Hl)[:, None]), dtype=bf)
    s2 = jnp.asarray(np.equal(np.arange(Wm)[:, None] // cpw,
                              np.arange(Wl)[None, :]),
                     dtype=np.dtype(dt.name) if hasattr(dt, 'name') else dt)

    G = _IMGS_PER_STEP if B % _IMGS_PER_STEP == 0 else 1
    out_shape = (jax.ShapeDtypeStruct((B, N, Hl, Wl), dt),
                 jax.ShapeDtypeStruct((B, N, Hl, Wl), dt))
    li, lm = pl.pallas_call(
        _extract_body,
        grid=(B // G,),
        in_specs=[
            pl.BlockSpec((G // 2, C, H, W), lambda b: (2 * b, 0, 0, 0)),
            pl.BlockSpec((G // 2, C, H, W), lambda b: (2 * b + 1, 0, 0, 0)),
            pl.BlockSpec((G, Hm, Wm), lambda b: (b, 0, 0)),
            pl.BlockSpec((N, C, _PATCH, W), lambda b: (0, 0, 0, 0)),
            pl.BlockSpec((W, Wm), lambda b: (0, 0)),
            pl.BlockSpec((Hm, H), lambda b: (0, 0)),
            pl.BlockSpec((Hl, Hm), lambda b: (0, 0)),
            pl.BlockSpec((Wm, Wl), lambda b: (0, 0)),
        ],
        out_specs=(pl.BlockSpec((G, N, Hl, Wl), lambda b: (b, 0, 0, 0)),
                   pl.BlockSpec((G, N, Hl, Wl), lambda b: (b, 0, 0, 0))),
        out_shape=out_shape,
        compiler_params=pltpu.CompilerParams(dimension_semantics=("parallel",)),
    )(images, images, ref_masks, wt, s4, rp4, rp2, s2)
    return li, lm


# speedup vs baseline: 1.0475x; 1.0475x over previous
import numpy as np
import jax
import jax.numpy as jnp
from jax.experimental import pallas as pl
from jax.experimental.pallas import tpu as pltpu

_PATCH = 8
_IMGS_PER_STEP = 4


def _extract_body(x_ref, m_ref, wt_ref, s4_ref, rp4_ref, rp2_ref,
                  s2_ref, oi_ref, om_ref):
    n_lat, n_ch = wt_ref.shape[0], wt_ref.shape[1]
    H, W = x_ref.shape[2], x_ref.shape[3]
    Wm = s4_ref.shape[1]
    Hm = rp4_ref.shape[0]
    Hl = rp2_ref.shape[0]
    hs = H // _PATCH

    for g in range(x_ref.shape[0]):
        xb = [x_ref[g, c].astype(jnp.bfloat16).reshape(hs, _PATCH, W)
              for c in range(n_ch)]
        mb = m_ref[g].astype(jnp.bfloat16)

        ts = []
        for n in range(n_lat):
            y = xb[0] * wt_ref[n, 0][None, :, :]
            for c in range(1, n_ch):
                y = y + xb[c] * wt_ref[n, c][None, :, :]
            ts.append(jnp.dot(rp4_ref[...], y.reshape(H, W),
                              preferred_element_type=jnp.float32))
        t4 = jnp.concatenate(ts, axis=0).astype(jnp.bfloat16)

        z = jnp.dot(t4, s4_ref[...],
                    preferred_element_type=jnp.float32)
        zb = z.astype(jnp.bfloat16)

        pieces = []
        for n in range(n_lat):
            zn = zb[n * Hm:(n + 1) * Hm]
            pieces.append(zn)
            pieces.append(zn * mb)
        tall = jnp.concatenate(pieces, axis=1)
        t2 = jnp.dot(rp2_ref[...], tall,
                     preferred_element_type=jnp.float32)

        s = jnp.concatenate(
            [t2[:, k * Wm:(k + 1) * Wm] for k in range(2 * n_lat)], axis=0)
        o = jnp.dot(s, s2_ref[...],
                    preferred_element_type=jnp.float32)

        for n in range(n_lat):
            oi_ref[g, n] = o[2 * n * Hl:(2 * n + 1) * Hl].astype(oi_ref.dtype)
            om_ref[g, n] = o[(2 * n + 1) * Hl:
                             (2 * n + 2) * Hl].astype(om_ref.dtype)


def kernel(images, ref_masks, w_kernel):
    B, C, H, W = images.shape
    _, Hm, Wm = ref_masks.shape
    N = w_kernel.shape[0]
    Hl, Wl = H // _PATCH, W // _PATCH
    qh, qw = H // Hm, W // Wm
    cph = _PATCH // qh
    cpw = _PATCH // qw
    dt = images.dtype
    bf = jnp.bfloat16

    t8 = np.equal(np.arange(W)[None, :] % _PATCH,
                  np.arange(_PATCH)[:, None]).astype(np.float32)
    wt = jnp.einsum('ncjd,dw->ncjw',
                    w_kernel.reshape(N, C, _PATCH, _PATCH), t8,
                    precision=jax.lax.Precision.HIGHEST).astype(bf)

    s4 = jnp.asarray(np.equal(np.arange(W)[:, None] // qw,
                              np.arange(Wm)[None, :]), dtype=bf)
    rp4 = jnp.asarray(np.equal(np.arange(H)[None, :] // qh,
                               np.arange(Hm)[:, None]), dtype=bf)
    rp2 = jnp.asarray(np.equal(np.arange(Hm)[None, :] // cph,
                               np.arange(Hl)[:, None]), dtype=bf)
    s2 = jnp.asarray(np.equal(np.arange(Wm)[:, None] // cpw,
                              np.arange(Wl)[None, :]),
                     dtype=np.dtype(dt.name) if hasattr(dt, 'name') else dt)

    G = _IMGS_PER_STEP if B % _IMGS_PER_STEP == 0 else 1
    out_shape = (jax.ShapeDtypeStruct((B, N, Hl, Wl), dt),
                 jax.ShapeDtypeStruct((B, N, Hl, Wl), dt))
    li, lm = pl.pallas_call(
        _extract_body,
        grid=(B // G,),
        in_specs=[
            pl.BlockSpec((G, C, H, W), lambda b: (b, 0, 0, 0)),
            pl.BlockSpec((G, Hm, Wm), lambda b: (b, 0, 0)),
            pl.BlockSpec((N, C, _PATCH, W), lambda b: (0, 0, 0, 0)),
            pl.BlockSpec((W, Wm), lambda b: (0, 0)),
            pl.BlockSpec((Hm, H), lambda b: (0, 0)),
            pl.BlockSpec((Hl, Hm), lambda b: (0, 0)),
            pl.BlockSpec((Wm, Wl), lambda b: (0, 0)),
        ],
        out_specs=(pl.BlockSpec((G, N, Hl, Wl), lambda b: (b, 0, 0, 0)),
                   pl.BlockSpec((G, N, Hl, Wl), lambda b: (b, 0, 0, 0))),
        out_shape=out_shape,
        compiler_params=pltpu.CompilerParams(dimension_semantics=("parallel",)),
    )(images, ref_masks, wt, s4, rp4, rp2, s2)
    return li, lm
